# baseline (speedup 1.0000x reference)
import functools

import jax
import jax.numpy as jnp
from jax import lax
from jax.experimental import pallas as pl
from jax.experimental.pallas import tpu as pltpu
from jax.experimental.pallas import tpu_sc as plsc

B = 1024
D = 16
NUM_ENT = 1000000
NUM_TYPE = 1000
NUM_CORES = 2
NUM_SUBCORES = 16
NW = NUM_CORES * NUM_SUBCORES
B_PER_W = B // NW


def _sc_gather(ent, ent_type, ee1d, te1d, ep1d, tp1d):
    mesh = plsc.VectorSubcoreMesh(core_axis_name="c", subcore_axis_name="s")
    row_t = jax.ShapeDtypeStruct((B, D), jnp.float32)

    @functools.partial(
        pl.kernel,
        mesh=mesh,
        out_type=[row_t, row_t, row_t, row_t],
        scratch_types=[
            pltpu.VMEM((B_PER_W,), jnp.int32),
            pltpu.VMEM((B_PER_W,), jnp.int32),
            pltpu.VMEM((B_PER_W, D), jnp.float32),
            pltpu.VMEM((B_PER_W, D), jnp.float32),
            pltpu.VMEM((B_PER_W, D), jnp.float32),
            pltpu.VMEM((B_PER_W, D), jnp.float32),
            pltpu.SemaphoreType.DMA,
        ],
    )
    def gather_kernel(ent_hbm, etype_hbm, ee_hbm, te_hbm, ep_hbm, tp_hbm,
                      e_out, t_out, pe_out, pt_out,
                      idx_e, idx_t, e_v, t_v, pe_v, pt_v, sem):
        wid = lax.axis_index("s") * NUM_CORES + lax.axis_index("c")
        base = wid * B_PER_W
        sl = pl.ds(base, B_PER_W)
        pltpu.sync_copy(ent_hbm.at[sl], idx_e)
        pltpu.sync_copy(etype_hbm.at[sl], idx_t)
        lanes = lax.iota(jnp.int32, 16)
        off_ent = lanes * NUM_ENT
        off_type = lanes * NUM_TYPE
        copies = []
        for g in range(B_PER_W // 16):
            ve = idx_e[pl.ds(g * 16, 16)]
            vt = idx_t[pl.ds(g * 16, 16)]
            for k in range(16):
                r = g * 16 + k
                ae = off_ent + ve[k]
                at = off_type + vt[k]
                copies.append(pltpu.async_copy(ee_hbm.at[ae], e_v.at[r], sem))
                copies.append(pltpu.async_copy(ep_hbm.at[ae], pe_v.at[r], sem))
                copies.append(pltpu.async_copy(te_hbm.at[at], t_v.at[r], sem))
                copies.append(pltpu.async_copy(tp_hbm.at[at], pt_v.at[r], sem))
        for c in copies:
            c.wait()
        pltpu.sync_copy(e_v, e_out.at[sl])
        pltpu.sync_copy(pe_v, pe_out.at[sl])
        pltpu.sync_copy(t_v, t_out.at[sl])
        pltpu.sync_copy(pt_v, pt_out.at[sl])

    return gather_kernel(ent, ent_type, ee1d, te1d, ep1d, tp1d)


BI = 256


def _score_body(e_ref, t_ref, pet_ref, ptt_ref, out_ref):
    e = e_ref[...]
    t = t_ref[...]
    pet = pet_ref[...]
    ptt = ptt_ref[...]
    a = jax.lax.dot_general(e, pet, (((1,), (0,)), ((), ())),
                            preferred_element_type=jnp.float32,
                            precision=jax.lax.Precision.HIGHEST)
    c = jax.lax.dot_general(t, ptt, (((1,), (0,)), ((), ())),
                            preferred_element_type=jnp.float32,
                            precision=jax.lax.Precision.HIGHEST)
    diff = e - t
    acc = jnp.zeros((BI, B), jnp.float32)
    for d in range(D):
        term = diff[:, d:d + 1] + a * pet[d:d + 1, :] - c * ptt[d:d + 1, :]
        acc = acc + jnp.abs(term)
    out_ref[...] = acc


def _tc_score(e, t, pe_t, pt_t):
    return pl.pallas_call(
        _score_body,
        grid=(B // BI,),
        in_specs=[
            pl.BlockSpec((BI, D), lambda i: (i, 0)),
            pl.BlockSpec((BI, D), lambda i: (i, 0)),
            pl.BlockSpec((D, B), lambda i: (0, 0)),
            pl.BlockSpec((D, B), lambda i: (0, 0)),
        ],
        out_specs=pl.BlockSpec((BI, B), lambda i: (i, 0)),
        out_shape=jax.ShapeDtypeStruct((B, B), jnp.float32),
    )(e, t, pe_t, pt_t)


def kernel(ent, ent_type, ent_emb, type_emb, ent_proj, type_proj):
    e, t, pe, pt = _sc_gather(
        ent, ent_type,
        ent_emb.T.reshape(-1), type_emb.T.reshape(-1),
        ent_proj.T.reshape(-1), type_proj.T.reshape(-1))
    return _tc_score(e, t, pe.T, pt.T)


# speedup vs baseline: 40.8963x; 40.8963x over previous
import functools

import jax
import jax.numpy as jnp
from jax import lax
from jax.experimental import pallas as pl
from jax.experimental.pallas import tpu as pltpu
from jax.experimental.pallas import tpu_sc as plsc

B = 1024
D = 16
NUM_ENT = 1000000
NUM_TYPE = 1000
NUM_CORES = 2
NUM_SUBCORES = 16
NW = NUM_CORES * NUM_SUBCORES
B_PER_W = B // NW
SLAB = 128
GRP = 16


def _sc_gather(ent, ent_type, ee_t, te_t, ep_t, tp_t):
    mesh = plsc.VectorSubcoreMesh(core_axis_name="c", subcore_axis_name="s")
    blk_t = jax.ShapeDtypeStruct((NW, D, B_PER_W), jnp.float32)

    @functools.partial(
        pl.kernel,
        mesh=mesh,
        compiler_params=pltpu.CompilerParams(needs_layout_passes=False),
        out_type=[blk_t, blk_t, blk_t, blk_t],
        scratch_types=[
            pltpu.VMEM((B_PER_W,), jnp.int32),
            pltpu.VMEM((B_PER_W,), jnp.int32),
            pltpu.VMEM((GRP, D, SLAB), jnp.float32),
            pltpu.VMEM((GRP, D, SLAB), jnp.float32),
            pltpu.VMEM((D, NUM_TYPE), jnp.float32),
            pltpu.VMEM((D, NUM_TYPE), jnp.float32),
            pltpu.VMEM((D, B_PER_W), jnp.float32),
            pltpu.VMEM((D, B_PER_W), jnp.float32),
            pltpu.VMEM((D, B_PER_W), jnp.float32),
            pltpu.VMEM((D, B_PER_W), jnp.float32),
            pltpu.SemaphoreType.DMA,
            pltpu.SemaphoreType.DMA,
        ],
    )
    def gather_kernel(ent_hbm, etype_hbm, ee_hbm, te_hbm, ep_hbm, tp_hbm,
                      e_out, t_out, pe_out, pt_out,
                      idx_e, idx_t, slab_e, slab_p, ty_e, ty_p,
                      e_v, t_v, pe_v, pt_v, sem, sem_ty):
        wid = lax.axis_index("s") * NUM_CORES + lax.axis_index("c")
        base = wid * B_PER_W
        sl = pl.ds(base, B_PER_W)
        pltpu.sync_copy(ent_hbm.at[sl], idx_e)
        pltpu.sync_copy(etype_hbm.at[sl], idx_t)
        ct_e = pltpu.async_copy(te_hbm, ty_e, sem_ty)
        ct_p = pltpu.async_copy(tp_hbm, ty_p, sem_ty)
        grp16 = lax.iota(jnp.int32, GRP)

        for g in range(B_PER_W // GRP):
            gsl = pl.ds(g * GRP, GRP)
            ve = idx_e[gsl]
            lane_v = ve % SLAB
            cps = []
            for k in range(GRP):
                ie = ve[k]
                col = pl.multiple_of(ie - lax.rem(ie, SLAB), SLAB)
                cps.append(pltpu.async_copy(
                    ee_hbm.at[:, pl.ds(col, SLAB)], slab_e.at[k], sem))
                cps.append(pltpu.async_copy(
                    ep_hbm.at[:, pl.ds(col, SLAB)], slab_p.at[k], sem))
            for c in cps:
                c.wait()
            for d in range(D):
                d_v = jnp.full((GRP,), d, jnp.int32)
                e_v[d, gsl] = plsc.load_gather(slab_e, [grp16, d_v, lane_v])
                pe_v[d, gsl] = plsc.load_gather(slab_p, [grp16, d_v, lane_v])

        ct_e.wait()
        ct_p.wait()
        for g in range(B_PER_W // GRP):
            gsl = pl.ds(g * GRP, GRP)
            vt = idx_t[gsl]
            for d in range(D):
                d_v = jnp.full((GRP,), d, jnp.int32)
                t_v[d, gsl] = plsc.load_gather(ty_e, [d_v, vt])
                pt_v[d, gsl] = plsc.load_gather(ty_p, [d_v, vt])

        pltpu.sync_copy(e_v, e_out.at[wid])
        pltpu.sync_copy(pe_v, pe_out.at[wid])
        pltpu.sync_copy(t_v, t_out.at[wid])
        pltpu.sync_copy(pt_v, pt_out.at[wid])

    return gather_kernel(ent, ent_type, ee_t, te_t, ep_t, tp_t)


BI = 256


def _score_body(e_ref, t_ref, pet_ref, ptt_ref, out_ref):
    e = e_ref[...]
    t = t_ref[...]
    pet = pet_ref[...]
    ptt = ptt_ref[...]
    a = jax.lax.dot_general(e, pet, (((1,), (0,)), ((), ())),
                            preferred_element_type=jnp.float32,
                            precision=jax.lax.Precision.HIGHEST)
    c = jax.lax.dot_general(t, ptt, (((1,), (0,)), ((), ())),
                            preferred_element_type=jnp.float32,
                            precision=jax.lax.Precision.HIGHEST)
    diff = e - t
    acc = jnp.zeros((BI, B), jnp.float32)
    for d in range(D):
        term = diff[:, d:d + 1] + a * pet[d:d + 1, :] - c * ptt[d:d + 1, :]
        acc = acc + jnp.abs(term)
    out_ref[...] = acc


def _tc_score(e, t, pe_t, pt_t):
    return pl.pallas_call(
        _score_body,
        grid=(B // BI,),
        in_specs=[
            pl.BlockSpec((BI, D), lambda i: (i, 0)),
            pl.BlockSpec((BI, D), lambda i: (i, 0)),
            pl.BlockSpec((D, B), lambda i: (0, 0)),
            pl.BlockSpec((D, B), lambda i: (0, 0)),
        ],
        out_specs=pl.BlockSpec((BI, B), lambda i: (i, 0)),
        out_shape=jax.ShapeDtypeStruct((B, B), jnp.float32),
    )(e, t, pe_t, pt_t)


def kernel(ent, ent_type, ent_emb, type_emb, ent_proj, type_proj):
    e_b, t_b, pe_b, pt_b = _sc_gather(
        ent, ent_type,
        ent_emb.T, type_emb.T, ent_proj.T, type_proj.T)
    e = e_b.transpose(0, 2, 1).reshape(B, D)
    t = t_b.transpose(0, 2, 1).reshape(B, D)
    pe_t = pe_b.transpose(1, 0, 2).reshape(D, B)
    pt_t = pt_b.transpose(1, 0, 2).reshape(D, B)
    return _tc_score(e, t, pe_t, pt_t)
